# baseline (speedup 1.0000x reference)
import jax
import jax.numpy as jnp
from jax import lax
from jax.experimental import pallas as pl
from jax.experimental.pallas import tpu as pltpu
from jax.experimental.pallas import tpu_sc as plsc

VOCAB = 1000000
EMBED = 32
BATCH = 16384

NUM_CORES = 2
NUM_SUBCORES = 16
NUM_WORKERS = NUM_CORES * NUM_SUBCORES
B_PER_W = BATCH // NUM_WORKERS
CHUNK = 128
NCHUNK = B_PER_W // CHUNK
GROUPS = B_PER_W // 16


def _sc_body(x_hbm, t_hbm, wu_hbm, wv_hbm, out_hbm,
             xidx, tidx, urows, vrows, outv, sem_u, sem_v):
    c = lax.axis_index("c")
    s = lax.axis_index("s")
    wid = s * NUM_CORES + c
    base = wid * B_PER_W

    pltpu.sync_copy(x_hbm.at[pl.ds(base, B_PER_W)], xidx)
    pltpu.sync_copy(t_hbm.at[pl.ds(base, B_PER_W)], tidx)

    copies = []
    for ch in range(NCHUNK):
        dst = pl.ds(ch * CHUNK, CHUNK)
        copies.append(pltpu.async_copy(
            wu_hbm.at[xidx.at[pl.ds(ch * CHUNK, CHUNK)]], urows.at[dst], sem_u))
        copies.append(pltpu.async_copy(
            wv_hbm.at[tidx.at[pl.ds(ch * CHUNK, CHUNK)]], vrows.at[dst], sem_v))
    for cp in copies:
        cp.wait()

    lanes = lax.iota(jnp.int32, 16)

    def group_body(g, carry):
        rows = g * 16 + lanes
        acc = jnp.zeros((16,), jnp.float32)
        for j in range(EMBED):
            col = jnp.full((16,), j, jnp.int32)
            uv = plsc.load_gather(urows, [rows, col])
            vv = plsc.load_gather(vrows, [rows, col])
            acc = acc + uv * vv
        outv[pl.ds(g * 16, 16)] = acc
        return carry

    lax.fori_loop(0, GROUPS, group_body, 0)

    pltpu.sync_copy(outv, out_hbm.at[pl.ds(base, B_PER_W)])


@jax.jit
def _run(x, t, W_u, W_v):
    eye = jnp.eye(EMBED, dtype=jnp.float32)
    W_u = lax.optimization_barrier(W_u) @ eye
    W_v = lax.optimization_barrier(W_v) @ eye
    mesh = plsc.VectorSubcoreMesh(core_axis_name="c", subcore_axis_name="s")
    kfn = pl.kernel(
        _sc_body,
        out_type=jax.ShapeDtypeStruct((BATCH,), jnp.float32),
        mesh=mesh,
        scratch_types=[
            pltpu.VMEM((B_PER_W,), jnp.int32),
            pltpu.VMEM((B_PER_W,), jnp.int32),
            pltpu.VMEM((B_PER_W, EMBED), jnp.float32),
            pltpu.VMEM((B_PER_W, EMBED), jnp.float32),
            pltpu.VMEM((B_PER_W,), jnp.float32),
            pltpu.SemaphoreType.DMA,
            pltpu.SemaphoreType.DMA,
        ],
        compiler_params=pltpu.CompilerParams(
            needs_layout_passes=False, use_tc_tiling_on_sc=False),
    )
    return kfn(x, t, W_u, W_v)


def kernel(x, t, W_u, W_v):
    return _run(x.astype(jnp.int32), t.astype(jnp.int32), W_u, W_v)


# speedup vs baseline: 1.8460x; 1.8460x over previous
import jax
import jax.numpy as jnp
from jax import lax
from jax.experimental import pallas as pl
from jax.experimental.pallas import tpu as pltpu
from jax.experimental.pallas import tpu_sc as plsc

VOCAB = 1000000
EMBED = 32
BATCH = 16384

NUM_CORES = 2
NUM_SUBCORES = 16
NUM_WORKERS = NUM_CORES * NUM_SUBCORES
B_PER_W = BATCH // NUM_WORKERS
PASS_ROWS = 256
NPASS = B_PER_W // PASS_ROWS
PGROUPS = PASS_ROWS // 16


def _sc_body(x_hbm, t_hbm, wu_hbm, wv_hbm, out_hbm,
             xidx, tidx, urows, vrows, outv, sem_u, sem_v):
    c = lax.axis_index("c")
    s = lax.axis_index("s")
    wid = s * NUM_CORES + c
    base = wid * B_PER_W

    pltpu.sync_copy(x_hbm.at[pl.ds(base, B_PER_W)], xidx)
    pltpu.sync_copy(t_hbm.at[pl.ds(base, B_PER_W)], tidx)

    lane = lax.iota(jnp.int32, 16)
    perms = {st: lane ^ st for st in (1, 2, 4, 8)}
    masks = {st: (lane & st) != 0 for st in (1, 2, 4, 8)}

    def pass_body(p, carry):
        poff = p * PASS_ROWS

        def fire_body(g, carry):
            xv = xidx[pl.ds(poff + g * 16, 16)]
            tv = tidx[pl.ds(poff + g * 16, 16)]
            for j in range(16):
                row = g * 16 + j
                pltpu.async_copy(wu_hbm.at[pl.ds(xv[j], 1), :],
                                 urows.at[pl.ds(row, 1), :], sem_u)
                pltpu.async_copy(wv_hbm.at[pl.ds(tv[j], 1), :],
                                 vrows.at[pl.ds(row, 1), :], sem_v)
            return carry

        lax.fori_loop(0, PGROUPS, fire_body, 0)

        pltpu.make_async_copy(wu_hbm.at[pl.ds(0, PASS_ROWS), :],
                              urows, sem_u).wait()
        pltpu.make_async_copy(wv_hbm.at[pl.ds(0, PASS_ROWS), :],
                              vrows, sem_v).wait()

        def group_body(g, carry):
            vecs = []
            for i in range(16):
                row = g * 16 + i
                u0 = urows[row, pl.ds(0, 16)]
                u1 = urows[row, pl.ds(16, 16)]
                v0 = vrows[row, pl.ds(0, 16)]
                v1 = vrows[row, pl.ds(16, 16)]
                vecs.append(u0 * v0 + u1 * v1)
            for st in (1, 2, 4, 8):
                nxt = []
                for k in range(len(vecs) // 2):
                    xk, yk = vecs[2 * k], vecs[2 * k + 1]
                    dx = xk + xk.at[perms[st]].get(mode="promise_in_bounds")
                    dy = yk + yk.at[perms[st]].get(mode="promise_in_bounds")
                    nxt.append(jnp.where(masks[st], dy, dx))
                vecs = nxt
            outv[pl.ds(poff + g * 16, 16)] = vecs[0]
            return carry

        lax.fori_loop(0, PGROUPS, group_body, 0)
        return carry

    lax.fori_loop(0, NPASS, pass_body, 0)

    pltpu.sync_copy(outv, out_hbm.at[pl.ds(base, B_PER_W)])


@jax.jit
def _run(x, t, W_u, W_v):
    mesh = plsc.VectorSubcoreMesh(core_axis_name="c", subcore_axis_name="s")
    kfn = pl.kernel(
        _sc_body,
        out_type=jax.ShapeDtypeStruct((BATCH,), jnp.float32),
        mesh=mesh,
        scratch_types=[
            pltpu.VMEM((B_PER_W,), jnp.int32),
            pltpu.VMEM((B_PER_W,), jnp.int32),
            pltpu.VMEM((PASS_ROWS, EMBED), jnp.float32),
            pltpu.VMEM((PASS_ROWS, EMBED), jnp.float32),
            pltpu.VMEM((B_PER_W,), jnp.float32),
            pltpu.SemaphoreType.DMA,
            pltpu.SemaphoreType.DMA,
        ],
        compiler_params=pltpu.CompilerParams(needs_layout_passes=False),
    )
    return kfn(x, t, W_u, W_v)


def kernel(x, t, W_u, W_v):
    return _run(x.astype(jnp.int32), t.astype(jnp.int32), W_u, W_v)


# speedup vs baseline: 1.8482x; 1.0012x over previous
import jax
import jax.numpy as jnp
from jax import lax
from jax.experimental import pallas as pl
from jax.experimental.pallas import tpu as pltpu
from jax.experimental.pallas import tpu_sc as plsc

VOCAB = 1000000
EMBED = 32
BATCH = 16384

NUM_CORES = 2
NUM_SUBCORES = 16
NUM_WORKERS = NUM_CORES * NUM_SUBCORES
B_PER_W = BATCH // NUM_WORKERS
PASS_ROWS = 256
NPASS = B_PER_W // PASS_ROWS
PGROUPS = PASS_ROWS // 16


def _sc_body(x_hbm, t_hbm, wu_hbm, wv_hbm, out_hbm,
             xidx, tidx, urows, vrows, outv, sem_u, sem_v):
    c = lax.axis_index("c")
    s = lax.axis_index("s")
    wid = s * NUM_CORES + c
    base = wid * B_PER_W

    pltpu.sync_copy(x_hbm.at[pl.ds(base, B_PER_W)], xidx)
    pltpu.sync_copy(t_hbm.at[pl.ds(base, B_PER_W)], tidx)

    lane = lax.iota(jnp.int32, 16)
    perms = {st: lane ^ st for st in (1, 2, 4, 8)}
    masks = {st: (lane & st) != 0 for st in (1, 2, 4, 8)}

    def pass_body(p, carry):
        poff = p * PASS_ROWS

        def fire_body(g, carry):
            xv = xidx[pl.ds(poff + g * 16, 16)]
            tv = tidx[pl.ds(poff + g * 16, 16)]
            for j in range(16):
                row = g * 16 + j
                pltpu.async_copy(wu_hbm.at[pl.ds(xv[j], 1), :],
                                 urows.at[pl.ds(row, 1), :], sem_u)
                pltpu.async_copy(wv_hbm.at[pl.ds(tv[j], 1), :],
                                 vrows.at[pl.ds(row, 1), :], sem_v)
            return carry

        lax.fori_loop(0, PGROUPS, fire_body, 0)

        pltpu.make_async_copy(wu_hbm.at[pl.ds(0, PASS_ROWS), :],
                              urows, sem_u).wait()
        pltpu.make_async_copy(wv_hbm.at[pl.ds(0, PASS_ROWS), :],
                              vrows, sem_v).wait()

        def group_body(g, carry):
            vecs = []
            for i in range(16):
                row = g * 16 + i
                u0 = urows[row, pl.ds(0, 16)]
                u1 = urows[row, pl.ds(16, 16)]
                v0 = vrows[row, pl.ds(0, 16)]
                v1 = vrows[row, pl.ds(16, 16)]
                vecs.append(u0 * v0 + u1 * v1)
            for st in (1, 2, 4, 8):
                nxt = []
                for k in range(len(vecs) // 2):
                    xk, yk = vecs[2 * k], vecs[2 * k + 1]
                    dx = xk + xk.at[perms[st]].get(mode="promise_in_bounds")
                    dy = yk + yk.at[perms[st]].get(mode="promise_in_bounds")
                    nxt.append(jnp.where(masks[st], dy, dx))
                vecs = nxt
            outv[pl.ds(poff + g * 16, 16)] = vecs[0]
            return carry

        lax.fori_loop(0, PGROUPS, group_body, 0)
        return carry

    lax.fori_loop(0, NPASS, pass_body, 0)

    pltpu.sync_copy(outv, out_hbm.at[pl.ds(base, B_PER_W)])


@jax.jit
def _run(x, t, W_u, W_v):
    mesh = plsc.VectorSubcoreMesh(core_axis_name="c", subcore_axis_name="s")
    kfn = pl.kernel(
        _sc_body,
        out_type=jax.ShapeDtypeStruct((BATCH,), jnp.float32),
        mesh=mesh,
        scratch_types=[
            pltpu.VMEM((B_PER_W,), jnp.int32),
            pltpu.VMEM((B_PER_W,), jnp.int32),
            pltpu.VMEM((PASS_ROWS, EMBED), jnp.float32),
            pltpu.VMEM((PASS_ROWS, EMBED), jnp.float32),
            pltpu.VMEM((B_PER_W,), jnp.float32),
            pltpu.SemaphoreType.DMA,
            pltpu.SemaphoreType.DMA,
        ],
        compiler_params=pltpu.CompilerParams(
            needs_layout_passes=False,
            skip_device_barrier=True,
            disable_bounds_checks=True,
            disable_semaphore_checks=True,
        ),
    )
    return kfn(x, t, W_u, W_v)


def kernel(x, t, W_u, W_v):
    return _run(x.astype(jnp.int32), t.astype(jnp.int32), W_u, W_v)
